# baseline (speedup 1.0000x reference)
import functools

import jax
import jax.numpy as jnp
from jax import lax
from jax.experimental import pallas as pl
from jax.experimental.pallas import tpu as pltpu
from jax.experimental.pallas import tpu_sc as plsc

N = 10000
E = 320000
D = 128
F2 = 8
G = 64

NC = 2
NS = 16
L = 16
NW = NC * NS
N_PAD = 10240
RPT = N_PAD // NS
CH = 64
NCH = 160
NB = 80
E_PAD = NW * NCH * CH
R = 1000


def _mesh():
    return plsc.VectorSubcoreMesh(core_axis_name="c", subcore_axis_name="s")


def _deg_body(dst_hbm, out_hbm, idx_v, ones_v, zb_v, acc_sh):
    c = lax.axis_index("c")
    s = lax.axis_index("s")
    wid = c * NS + s
    zeros16 = jnp.zeros((L,), jnp.float32)
    ones16 = jnp.ones((L,), jnp.float32)

    def fill(i, carry):
        zb_v[pl.ds(i * L, L)] = zeros16
        ones_v[pl.ds((i % (CH // L)) * L, L)] = ones16
        return carry

    lax.fori_loop(0, RPT // L, fill, 0)
    pltpu.sync_copy(zb_v, acc_sh.at[pl.ds(s * RPT, RPT)])
    pltpu.sync_copy(dst_hbm.at[pl.ds(wid * NCH, NCH)], idx_v)
    plsc.subcore_barrier()

    def body(j, carry):
        pltpu.sync_copy(ones_v, acc_sh.at[idx_v.at[j]], add=True)
        return carry

    lax.fori_loop(0, NCH, body, 0)
    plsc.subcore_barrier()
    pltpu.sync_copy(acc_sh.at[pl.ds(s * RPT, RPT)],
                    out_hbm.at[c, pl.ds(s * RPT, RPT)])


_deg_call = functools.partial(
    pl.kernel,
    out_type=jax.ShapeDtypeStruct((NC, N_PAD), jnp.float32),
    mesh=_mesh(),
    scratch_types=[
        pltpu.VMEM((NCH, CH), jnp.int32),
        pltpu.VMEM((CH,), jnp.float32),
        pltpu.VMEM((RPT,), jnp.float32),
        pltpu.VMEM_SHARED((N_PAD,), jnp.float32),
    ],
)(_deg_body)


RSTG = N // NS


def _make_scatter(feat, staged, async_scatter, lead, nbuf):
    NBUF = nbuf
    NG = NCH // NBUF
    def slot(j, b, zeros_hbm, src_ref, src_v, dst_v, buf, acc_sh, gsem, ssem,
             do_wait, do_issue):
        bg = (b + lead) % NBUF
        if do_issue:
            if do_wait and async_scatter:
                pltpu.make_async_copy(
                    zeros_hbm, buf.at[bg], ssem.at[bg]).wait()
            pltpu.async_copy(
                src_ref.at[src_v.at[j + lead]], buf.at[bg], gsem.at[bg])
        pltpu.make_async_copy(
            src_ref.at[src_v.at[j]], buf.at[b], gsem.at[b]).wait()
        if async_scatter:
            pltpu.async_copy(
                buf.at[b], acc_sh.at[dst_v.at[j]], ssem.at[b], add=True)
        else:
            pltpu.sync_copy(buf.at[b], acc_sh.at[dst_v.at[j]], add=True)

    def body(xs_hbm, src_hbm, dst_hbm, zeros_hbm, out_hbm, src_v, dst_v,
             buf, acc_sh, *rest):
        if staged and async_scatter:
            xs_sh, gsem, ssem = rest
        elif staged:
            xs_sh, gsem = rest
            ssem = None
        elif async_scatter:
            gsem, ssem = rest
        else:
            (gsem,) = rest
            ssem = None
        c = lax.axis_index("c")
        s = lax.axis_index("s")
        wid = c * NS + s

        def zs(t, carry):
            pltpu.sync_copy(zeros_hbm, acc_sh.at[pl.ds(s * RPT + t * CH, CH)])
            return carry

        lax.fori_loop(0, RPT // CH, zs, 0)
        if staged:
            pltpu.sync_copy(xs_hbm.at[pl.ds(s * RSTG, RSTG)],
                            xs_sh.at[pl.ds(s * RSTG, RSTG)])
        pltpu.sync_copy(src_hbm.at[pl.ds(wid * NCH, NCH)], src_v)
        pltpu.sync_copy(dst_hbm.at[pl.ds(wid * NCH, NCH)], dst_v)
        plsc.subcore_barrier()

        src_ref = xs_sh if staged else xs_hbm
        for b in range(lead):
            pltpu.async_copy(src_ref.at[src_v.at[b]], buf.at[b], gsem.at[b])
        for b in range(NBUF):
            slot(b, b, zeros_hbm, src_ref, src_v, dst_v, buf, acc_sh, gsem,
                 ssem, do_wait=(b >= NBUF - lead), do_issue=True)

        def grp(g, carry):
            for b in range(NBUF):
                slot(g * NBUF + b, b, zeros_hbm, src_ref, src_v, dst_v, buf,
                     acc_sh, gsem, ssem, do_wait=True, do_issue=True)
            return carry

        lax.fori_loop(1, NG - 1, grp, 0)
        for b in range(NBUF):
            slot((NG - 1) * NBUF + b, b, zeros_hbm, src_ref, src_v, dst_v,
                 buf, acc_sh, gsem, ssem, do_wait=(b < lead),
                 do_issue=(b < lead))
        if async_scatter:
            for b in range(NBUF):
                pltpu.make_async_copy(zeros_hbm, buf.at[b], ssem.at[b]).wait()
        plsc.subcore_barrier()

        def co(t, carry):
            pltpu.sync_copy(acc_sh.at[pl.ds(s * RPT + t * CH, CH)],
                            out_hbm.at[c, pl.ds(s * RPT + t * CH, CH)])
            return carry

        lax.fori_loop(0, RPT // CH, co, 0)

    scratch = [
        pltpu.VMEM((NCH, CH), jnp.int32),
        pltpu.VMEM((NCH, CH), jnp.int32),
        pltpu.VMEM((NBUF, CH, feat), jnp.float32),
        pltpu.VMEM_SHARED((N_PAD, feat), jnp.float32),
    ]
    if staged:
        scratch.append(pltpu.VMEM_SHARED((N_PAD, feat), jnp.float32))
    scratch.append(pltpu.SemaphoreType.DMA((NBUF,)))
    if async_scatter:
        scratch.append(pltpu.SemaphoreType.DMA((NBUF,)))
    return pl.kernel(
        body,
        out_type=jax.ShapeDtypeStruct((NC, N_PAD, feat), jnp.float32),
        mesh=_mesh(),
        compiler_params=pltpu.CompilerParams(use_tc_tiling_on_sc=False),
        scratch_types=scratch,
    )


NBUF128 = 4
LEAD128 = 3
NG128 = NB // NBUF128


def _scat128_body(xs_hbm, src_hbm, dst_hbm, zeros_hbm, out_hbm, src_v, dst_v,
                  b0, b1, b2, b3, acc_sh, s0, s1, s2, s3):
    c = lax.axis_index("c")
    s = lax.axis_index("s")
    wid = (1 - c) * NS + s
    bufs = [b0, b1, b2, b3]
    sems = [s0, s1, s2, s3]

    def zs(t, carry):
        pltpu.sync_copy(zeros_hbm, acc_sh.at[pl.ds(s * RPT + t * CH, CH)])
        return carry

    lax.fori_loop(0, RPT // CH, zs, 0)
    plsc.subcore_barrier()

    def slot(j, b, do_issue):
        bg = (b + LEAD128) % NBUF128
        if do_issue:
            pltpu.async_copy(
                xs_hbm.at[src_v.at[j + LEAD128]], bufs[bg], sems[bg])
        pltpu.make_async_copy(xs_hbm.at[src_v.at[j]], bufs[b], sems[b]).wait()
        pltpu.sync_copy(bufs[b], acc_sh.at[dst_v.at[j]], add=True)

    def grp(g, carry):
        for b in range(NBUF128):
            slot(g * NBUF128 + b, b, do_issue=True)
        return carry

    for h in range(NCH // NB):
        pltpu.sync_copy(src_hbm.at[pl.ds(wid * NCH + h * NB, NB)], src_v)
        pltpu.sync_copy(dst_hbm.at[pl.ds(wid * NCH + h * NB, NB)], dst_v)
        for b in range(LEAD128):
            pltpu.async_copy(xs_hbm.at[src_v.at[b]], bufs[b], sems[b])
        lax.fori_loop(0, NG128 - 1, grp, 0)
        for b in range(NBUF128):
            slot((NG128 - 1) * NBUF128 + b, b,
                 do_issue=(b < NBUF128 - LEAD128))
    plsc.subcore_barrier()

    def co(t, carry):
        pltpu.sync_copy(acc_sh.at[pl.ds(s * RPT + t * CH, CH)],
                        out_hbm.at[c, pl.ds(s * RPT + t * CH, CH)])
        return carry

    lax.fori_loop(0, RPT // CH, co, 0)


_scat128_call = pl.kernel(
    _scat128_body,
    out_type=jax.ShapeDtypeStruct((NC, N_PAD, D), jnp.float32),
    mesh=_mesh(),
    compiler_params=pltpu.CompilerParams(use_tc_tiling_on_sc=False),
    scratch_types=[
        pltpu.VMEM((NB, CH), jnp.int32),
        pltpu.VMEM((NB, CH), jnp.int32),
        pltpu.VMEM((CH, D), jnp.float32),
        pltpu.VMEM((CH, D), jnp.float32),
        pltpu.VMEM((CH, D), jnp.float32),
        pltpu.VMEM((CH, D), jnp.float32),
        pltpu.VMEM_SHARED((N_PAD, D), jnp.float32),
        pltpu.SemaphoreType.DMA,
        pltpu.SemaphoreType.DMA,
        pltpu.SemaphoreType.DMA,
        pltpu.SemaphoreType.DMA,
    ],
)


_scat8_call = _make_scatter(F2, staged=True, async_scatter=True, lead=4,
                            nbuf=8)


def _k2_body(x_ref, w_ref, deg_ref, xs_ref, dinv_ref):
    degv = deg_ref[:, 0] + deg_ref[:, 1] + 1.0
    dinv = lax.rsqrt(degv)
    xw = jnp.dot(x_ref[...], w_ref[...], preferred_element_type=jnp.float32)
    xs_ref[...] = xw * dinv[:, None]
    dinv_ref[...] = dinv[:, None]


_k2_call = pl.pallas_call(
    _k2_body,
    grid=(N // R,),
    in_specs=[
        pl.BlockSpec((R, D), lambda i: (i, 0)),
        pl.BlockSpec((D, D), lambda i: (0, 0)),
        pl.BlockSpec((R, NC), lambda i: (i, 0)),
    ],
    out_specs=[
        pl.BlockSpec((R, D), lambda i: (i, 0)),
        pl.BlockSpec((R, 1), lambda i: (i, 0)),
    ],
    out_shape=[
        jax.ShapeDtypeStruct((N, D), jnp.float32),
        jax.ShapeDtypeStruct((N, 1), jnp.float32),
    ],
)


def _k4_body(acc_ref, xs1_ref, dinv_ref, b1_ref, w2_ref, xs2_ref):
    dinv = dinv_ref[...]
    pre = dinv * (acc_ref[0] + acc_ref[1] + xs1_ref[...]) + b1_ref[...]
    h1 = jnp.where(pre >= 0, pre, 0.2 * pre)
    y = jnp.dot(h1, w2_ref[...], preferred_element_type=jnp.float32)
    xs2_ref[...] = y * dinv


_k4_call = pl.pallas_call(
    _k4_body,
    grid=(N // R,),
    in_specs=[
        pl.BlockSpec((NC, R, D), lambda i: (0, i, 0)),
        pl.BlockSpec((R, D), lambda i: (i, 0)),
        pl.BlockSpec((R, 1), lambda i: (i, 0)),
        pl.BlockSpec((1, D), lambda i: (0, 0)),
        pl.BlockSpec((D, F2), lambda i: (0, 0)),
    ],
    out_specs=pl.BlockSpec((R, F2), lambda i: (i, 0)),
    out_shape=jax.ShapeDtypeStruct((N, F2), jnp.float32),
)


def _k6_body(acc2_ref, xs2_ref, dinv_ref, b2_ref, batch_ref, l1w_ref,
             l1b_ref, l2w_ref, l2b_ref, out_ref):
    pre = (dinv_ref[...] * (acc2_ref[0, :N, :] + acc2_ref[1, :N, :]
                            + xs2_ref[...]) + b2_ref[...])
    h2 = jnp.where(pre >= 0, pre, 0.2 * pre)
    gids = lax.broadcasted_iota(jnp.int32, (G, N), 0)
    onehot = (gids == batch_ref[...]).astype(jnp.float32)
    sums = jnp.dot(onehot, h2, preferred_element_type=jnp.float32)
    cnt = jnp.sum(onehot, axis=1, keepdims=True)
    pooled = sums / jnp.maximum(cnt, 1.0)
    z0 = jnp.dot(pooled, l1w_ref[...],
                 preferred_element_type=jnp.float32) + l1b_ref[...]
    z = jnp.where(z0 >= 0, z0, 0.2 * z0)
    out_ref[...] = jnp.dot(z, l2w_ref[...],
                           preferred_element_type=jnp.float32) + l2b_ref[...]


_k6_call = pl.pallas_call(
    _k6_body,
    out_shape=jax.ShapeDtypeStruct((G, 2), jnp.float32),
)


def kernel(x, edge_index, batch, W1, b1, W2, b2, lin1_W, lin1_b, lin2_W,
           lin2_b):
    src = edge_index[0]
    dst = edge_index[1]
    pad = E_PAD - E
    src_p = jnp.concatenate(
        [src, jnp.zeros((pad,), jnp.int32)]).reshape(E_PAD // CH, CH)
    pad_dst = N + jnp.arange(pad, dtype=jnp.int32) % (N_PAD - N)
    dst_p = jnp.concatenate([dst, pad_dst]).reshape(E_PAD // CH, CH)

    deg = _deg_call(dst_p)
    xs1, dinv = _k2_call(x, W1, deg.T)
    z128 = jnp.zeros((CH, D), jnp.float32)
    z8 = jnp.zeros((CH, F2), jnp.float32)
    acc1 = _scat128_call(xs1, src_p, dst_p, z128)
    xs2 = _k4_call(acc1, xs1, dinv, b1.reshape(1, D), W2)
    acc2 = _scat8_call(xs2, src_p, dst_p, z8)
    out = _k6_call(acc2, xs2, dinv, b2.reshape(1, F2),
                   batch.reshape(1, N), lin1_W, lin1_b.reshape(1, F2),
                   lin2_W, lin2_b.reshape(1, 2))
    return out


# speedup vs baseline: 1.0668x; 1.0668x over previous
import functools

import jax
import jax.numpy as jnp
from jax import lax
from jax.experimental import pallas as pl
from jax.experimental.pallas import tpu as pltpu
from jax.experimental.pallas import tpu_sc as plsc

N = 10000
E = 320000
D = 128
F2 = 8
G = 64

NC = 2
NS = 16
L = 16
NW = NC * NS
N_PAD = 10240
RPT = N_PAD // NS
CH = 64
NCH = 160
NB = 80
E_PAD = NW * NCH * CH
R = 1000


def _mesh():
    return plsc.VectorSubcoreMesh(core_axis_name="c", subcore_axis_name="s")


def _deg_body(dst_hbm, out_hbm, idx_v, ones_v, zb_v, acc_sh):
    c = lax.axis_index("c")
    s = lax.axis_index("s")
    wid = c * NS + s
    zeros16 = jnp.zeros((L,), jnp.float32)
    ones16 = jnp.ones((L,), jnp.float32)

    def fill(i, carry):
        zb_v[pl.ds(i * L, L)] = zeros16
        ones_v[pl.ds((i % (CH // L)) * L, L)] = ones16
        return carry

    lax.fori_loop(0, RPT // L, fill, 0)
    pltpu.sync_copy(zb_v, acc_sh.at[pl.ds(s * RPT, RPT)])
    pltpu.sync_copy(dst_hbm.at[pl.ds(wid * NCH, NCH)], idx_v)
    plsc.subcore_barrier()

    def body(j, carry):
        pltpu.sync_copy(ones_v, acc_sh.at[idx_v.at[j]], add=True)
        return carry

    lax.fori_loop(0, NCH, body, 0)
    plsc.subcore_barrier()
    pltpu.sync_copy(acc_sh.at[pl.ds(s * RPT, RPT)],
                    out_hbm.at[c, pl.ds(s * RPT, RPT)])


_deg_call = functools.partial(
    pl.kernel,
    out_type=jax.ShapeDtypeStruct((NC, N_PAD), jnp.float32),
    mesh=_mesh(),
    scratch_types=[
        pltpu.VMEM((NCH, CH), jnp.int32),
        pltpu.VMEM((CH,), jnp.float32),
        pltpu.VMEM((RPT,), jnp.float32),
        pltpu.VMEM_SHARED((N_PAD,), jnp.float32),
    ],
)(_deg_body)


RSTG = N // NS


def _make_scatter(feat, staged, async_scatter, lead, nbuf):
    NBUF = nbuf
    NG = NCH // NBUF
    def slot(j, b, zeros_hbm, src_ref, src_v, dst_v, buf, acc_sh, gsem, ssem,
             do_wait, do_issue):
        bg = (b + lead) % NBUF
        if do_issue:
            if do_wait and async_scatter:
                pltpu.make_async_copy(
                    zeros_hbm, buf.at[bg], ssem.at[bg]).wait()
            pltpu.async_copy(
                src_ref.at[src_v.at[j + lead]], buf.at[bg], gsem.at[bg])
        pltpu.make_async_copy(
            src_ref.at[src_v.at[j]], buf.at[b], gsem.at[b]).wait()
        if async_scatter:
            pltpu.async_copy(
                buf.at[b], acc_sh.at[dst_v.at[j]], ssem.at[b], add=True)
        else:
            pltpu.sync_copy(buf.at[b], acc_sh.at[dst_v.at[j]], add=True)

    def body(xs_hbm, src_hbm, dst_hbm, zeros_hbm, out_hbm, src_v, dst_v,
             buf, acc_sh, *rest):
        if staged and async_scatter:
            xs_sh, gsem, ssem = rest
        elif staged:
            xs_sh, gsem = rest
            ssem = None
        elif async_scatter:
            gsem, ssem = rest
        else:
            (gsem,) = rest
            ssem = None
        c = lax.axis_index("c")
        s = lax.axis_index("s")
        wid = c * NS + s

        def zs(t, carry):
            pltpu.sync_copy(zeros_hbm, acc_sh.at[pl.ds(s * RPT + t * CH, CH)])
            return carry

        lax.fori_loop(0, RPT // CH, zs, 0)
        if staged:
            pltpu.sync_copy(xs_hbm.at[pl.ds(s * RSTG, RSTG)],
                            xs_sh.at[pl.ds(s * RSTG, RSTG)])
        pltpu.sync_copy(src_hbm.at[pl.ds(wid * NCH, NCH)], src_v)
        pltpu.sync_copy(dst_hbm.at[pl.ds(wid * NCH, NCH)], dst_v)
        plsc.subcore_barrier()

        src_ref = xs_sh if staged else xs_hbm
        for b in range(lead):
            pltpu.async_copy(src_ref.at[src_v.at[b]], buf.at[b], gsem.at[b])
        for b in range(NBUF):
            slot(b, b, zeros_hbm, src_ref, src_v, dst_v, buf, acc_sh, gsem,
                 ssem, do_wait=(b >= NBUF - lead), do_issue=True)

        def grp(g, carry):
            for b in range(NBUF):
                slot(g * NBUF + b, b, zeros_hbm, src_ref, src_v, dst_v, buf,
                     acc_sh, gsem, ssem, do_wait=True, do_issue=True)
            return carry

        lax.fori_loop(1, NG - 1, grp, 0)
        for b in range(NBUF):
            slot((NG - 1) * NBUF + b, b, zeros_hbm, src_ref, src_v, dst_v,
                 buf, acc_sh, gsem, ssem, do_wait=(b < lead),
                 do_issue=(b < lead))
        if async_scatter:
            for b in range(NBUF):
                pltpu.make_async_copy(zeros_hbm, buf.at[b], ssem.at[b]).wait()
        plsc.subcore_barrier()

        def co(t, carry):
            pltpu.sync_copy(acc_sh.at[pl.ds(s * RPT + t * CH, CH)],
                            out_hbm.at[c, pl.ds(s * RPT + t * CH, CH)])
            return carry

        lax.fori_loop(0, RPT // CH, co, 0)

    scratch = [
        pltpu.VMEM((NCH, CH), jnp.int32),
        pltpu.VMEM((NCH, CH), jnp.int32),
        pltpu.VMEM((NBUF, CH, feat), jnp.float32),
        pltpu.VMEM_SHARED((N_PAD, feat), jnp.float32),
    ]
    if staged:
        scratch.append(pltpu.VMEM_SHARED((N_PAD, feat), jnp.float32))
    scratch.append(pltpu.SemaphoreType.DMA((NBUF,)))
    if async_scatter:
        scratch.append(pltpu.SemaphoreType.DMA((NBUF,)))
    return pl.kernel(
        body,
        out_type=jax.ShapeDtypeStruct((NC, N_PAD, feat), jnp.float32),
        mesh=_mesh(),
        compiler_params=pltpu.CompilerParams(use_tc_tiling_on_sc=False),
        scratch_types=scratch,
    )


NBUF128 = 4
LEAD128 = 3
UCH = 20
UT = 16
UF = 13
CF = 1
NG128 = UCH // NBUF128


def _scat128_body(xs_hbm, src_hbm, dst_hbm, zeros_hbm, out_hbm, src_v, dst_v,
                  b0, b1, b2, b3, acc_sh, s0, s1, s2, s3):
    c = lax.axis_index("c")
    s = lax.axis_index("s")
    wid = c * NS + s
    bufs = [b0, b1, b2, b3]
    sems = [s0, s1, s2, s3]

    def zs(t, carry):
        pltpu.sync_copy(zeros_hbm, acc_sh.at[pl.ds(s * RPT + t * CH, CH)])
        return carry

    lax.fori_loop(0, RPT // CH, zs, 0)
    plsc.subcore_barrier()

    def slot(j, b, do_issue):
        bg = (b + LEAD128) % NBUF128
        if do_issue:
            pltpu.async_copy(
                xs_hbm.at[src_v.at[j + LEAD128]], bufs[bg], sems[bg])
        pltpu.make_async_copy(xs_hbm.at[src_v.at[j]], bufs[b], sems[b]).wait()
        pltpu.sync_copy(bufs[b], acc_sh.at[dst_v.at[j]], add=True)

    def grp(g, carry):
        for b in range(NBUF128):
            slot(g * NBUF128 + b, b, do_issue=True)
        return carry

    nun = jnp.where(c == CF, UF, UT - UF)
    base = jnp.where(c == CF, s * (UF * UCH),
                     NW // 2 * UF * UCH + s * ((UT - UF) * UCH))

    def unit(h, carry):
        off = base + h * UCH
        pltpu.sync_copy(src_hbm.at[pl.ds(off, UCH)], src_v)
        pltpu.sync_copy(dst_hbm.at[pl.ds(off, UCH)], dst_v)
        for b in range(LEAD128):
            pltpu.async_copy(xs_hbm.at[src_v.at[b]], bufs[b], sems[b])
        lax.fori_loop(0, NG128 - 1, grp, 0)
        for b in range(NBUF128):
            slot((NG128 - 1) * NBUF128 + b, b,
                 do_issue=(b < NBUF128 - LEAD128))
        return carry

    lax.fori_loop(0, nun, unit, 0)
    plsc.subcore_barrier()

    def co(t, carry):
        pltpu.sync_copy(acc_sh.at[pl.ds(s * RPT + t * CH, CH)],
                        out_hbm.at[c, pl.ds(s * RPT + t * CH, CH)])
        return carry

    lax.fori_loop(0, RPT // CH, co, 0)


_scat128_call = pl.kernel(
    _scat128_body,
    out_type=jax.ShapeDtypeStruct((NC, N_PAD, D), jnp.float32),
    mesh=_mesh(),
    compiler_params=pltpu.CompilerParams(use_tc_tiling_on_sc=False),
    scratch_types=[
        pltpu.VMEM((UCH, CH), jnp.int32),
        pltpu.VMEM((UCH, CH), jnp.int32),
        pltpu.VMEM((CH, D), jnp.float32),
        pltpu.VMEM((CH, D), jnp.float32),
        pltpu.VMEM((CH, D), jnp.float32),
        pltpu.VMEM((CH, D), jnp.float32),
        pltpu.VMEM_SHARED((N_PAD, D), jnp.float32),
        pltpu.SemaphoreType.DMA,
        pltpu.SemaphoreType.DMA,
        pltpu.SemaphoreType.DMA,
        pltpu.SemaphoreType.DMA,
    ],
)


_scat8_call = _make_scatter(F2, staged=True, async_scatter=True, lead=4,
                            nbuf=8)


def _k2_body(x_ref, w_ref, deg_ref, xs_ref, dinv_ref):
    degv = deg_ref[:, 0] + deg_ref[:, 1] + 1.0
    dinv = lax.rsqrt(degv)
    xw = jnp.dot(x_ref[...], w_ref[...], preferred_element_type=jnp.float32)
    xs_ref[...] = xw * dinv[:, None]
    dinv_ref[...] = dinv[:, None]


_k2_call = pl.pallas_call(
    _k2_body,
    grid=(N // R,),
    in_specs=[
        pl.BlockSpec((R, D), lambda i: (i, 0)),
        pl.BlockSpec((D, D), lambda i: (0, 0)),
        pl.BlockSpec((R, NC), lambda i: (i, 0)),
    ],
    out_specs=[
        pl.BlockSpec((R, D), lambda i: (i, 0)),
        pl.BlockSpec((R, 1), lambda i: (i, 0)),
    ],
    out_shape=[
        jax.ShapeDtypeStruct((N, D), jnp.float32),
        jax.ShapeDtypeStruct((N, 1), jnp.float32),
    ],
)


def _k4_body(acc_ref, xs1_ref, dinv_ref, b1_ref, w2_ref, xs2_ref):
    dinv = dinv_ref[...]
    pre = dinv * (acc_ref[0] + acc_ref[1] + xs1_ref[...]) + b1_ref[...]
    h1 = jnp.where(pre >= 0, pre, 0.2 * pre)
    y = jnp.dot(h1, w2_ref[...], preferred_element_type=jnp.float32)
    xs2_ref[...] = y * dinv


_k4_call = pl.pallas_call(
    _k4_body,
    grid=(N // R,),
    in_specs=[
        pl.BlockSpec((NC, R, D), lambda i: (0, i, 0)),
        pl.BlockSpec((R, D), lambda i: (i, 0)),
        pl.BlockSpec((R, 1), lambda i: (i, 0)),
        pl.BlockSpec((1, D), lambda i: (0, 0)),
        pl.BlockSpec((D, F2), lambda i: (0, 0)),
    ],
    out_specs=pl.BlockSpec((R, F2), lambda i: (i, 0)),
    out_shape=jax.ShapeDtypeStruct((N, F2), jnp.float32),
)


def _k6_body(acc2_ref, xs2_ref, dinv_ref, b2_ref, batch_ref, l1w_ref,
             l1b_ref, l2w_ref, l2b_ref, out_ref):
    pre = (dinv_ref[...] * (acc2_ref[0, :N, :] + acc2_ref[1, :N, :]
                            + xs2_ref[...]) + b2_ref[...])
    h2 = jnp.where(pre >= 0, pre, 0.2 * pre)
    gids = lax.broadcasted_iota(jnp.int32, (G, N), 0)
    onehot = (gids == batch_ref[...]).astype(jnp.float32)
    sums = jnp.dot(onehot, h2, preferred_element_type=jnp.float32)
    cnt = jnp.sum(onehot, axis=1, keepdims=True)
    pooled = sums / jnp.maximum(cnt, 1.0)
    z0 = jnp.dot(pooled, l1w_ref[...],
                 preferred_element_type=jnp.float32) + l1b_ref[...]
    z = jnp.where(z0 >= 0, z0, 0.2 * z0)
    out_ref[...] = jnp.dot(z, l2w_ref[...],
                           preferred_element_type=jnp.float32) + l2b_ref[...]


_k6_call = pl.pallas_call(
    _k6_body,
    out_shape=jax.ShapeDtypeStruct((G, 2), jnp.float32),
)


def kernel(x, edge_index, batch, W1, b1, W2, b2, lin1_W, lin1_b, lin2_W,
           lin2_b):
    src = edge_index[0]
    dst = edge_index[1]
    pad = E_PAD - E
    src_p = jnp.concatenate(
        [src, jnp.zeros((pad,), jnp.int32)]).reshape(E_PAD // CH, CH)
    pad_dst = N + jnp.arange(pad, dtype=jnp.int32) % (N_PAD - N)
    dst_p = jnp.concatenate([dst, pad_dst]).reshape(E_PAD // CH, CH)

    deg = _deg_call(dst_p)
    xs1, dinv = _k2_call(x, W1, deg.T)
    z128 = jnp.zeros((CH, D), jnp.float32)
    z8 = jnp.zeros((CH, F2), jnp.float32)
    acc1 = _scat128_call(xs1, src_p, dst_p, z128)
    xs2 = _k4_call(acc1, xs1, dinv, b1.reshape(1, D), W2)
    acc2 = _scat8_call(xs2, src_p, dst_p, z8)
    out = _k6_call(acc2, xs2, dinv, b2.reshape(1, F2),
                   batch.reshape(1, N), lin1_W, lin1_b.reshape(1, F2),
                   lin2_W, lin2_b.reshape(1, 2))
    return out
